# baseline (speedup 1.0000x reference)
import functools

import jax
import jax.numpy as jnp
from jax import lax
from jax.experimental import pallas as pl
from jax.experimental.pallas import tpu as pltpu
from jax.experimental.pallas import tpu_sc as plsc

WIN = 33


def _make_window_kernel(N, M, Q, D, EMB, NB):
    def body(*refs):
        a_refs = refs[0:4]
        h_refs = refs[4:20]
        wt_ref, b_ref, o_ref = refs[20], refs[21], refs[22]
        QH = Q + 32
        parts = []
        for s in range(4):
            parts.append(a_refs[s][...])
            for j in range(4):
                parts.append(h_refs[4 * s + j][...])
        f_all = jnp.concatenate(parts, axis=0)
        h = jnp.dot(f_all, wt_ref[...], preferred_element_type=jnp.float32)
        g = jnp.tanh(h + b_ref[...])
        gp = jnp.concatenate([g[s * QH:(s + 1) * QH] for s in range(4)],
                             axis=1)
        a = gp[0:Q + 8] + gp[8:Q + 16] + gp[16:Q + 24] + gp[24:Q + 32]
        bb = a[0:Q + 7] + a[1:Q + 8]
        c = bb[0:Q + 5] + bb[2:Q + 7]
        d = c[0:Q] + c[4:Q + 4]
        o_ref[...] = (d + gp[32:Q + 32]) * (1.0 / WIN)

    in_specs = []
    for s in range(4):
        in_specs.append(
            pl.BlockSpec((Q, D), functools.partial(
                lambda s_, i: (s_ * (M // Q) + i, 0), s)))
    NH8 = N // 8
    for s in range(4):
        for j in range(4):
            in_specs.append(
                pl.BlockSpec((8, D), functools.partial(
                    lambda s_, j_, i: (
                        lax.rem(s_ * (M // 8) + (i + 1) * (Q // 8) + j_, NH8),
                        0), s, j)))
    in_specs.append(pl.BlockSpec((D, EMB), lambda i: (0, 0)))
    in_specs.append(pl.BlockSpec((1, EMB), lambda i: (0, 0)))

    return pl.pallas_call(
        body,
        grid=(NB,),
        in_specs=in_specs,
        out_specs=pl.BlockSpec((Q, 128), lambda i: (i, 0)),
        out_shape=jax.ShapeDtypeStruct((M, 128), jnp.float32),
    )


def _make_sc_gather(B, M, NW, CH):
    mesh = plsc.VectorSubcoreMesh(core_axis_name="c", subcore_axis_name="s")

    BW = CH * 128

    @functools.partial(
        pl.kernel,
        mesh=mesh,
        out_type=jax.ShapeDtypeStruct((B, 128), jnp.float32),
        scratch_types=[
            pltpu.VMEM((BW,), jnp.int32),
            pltpu.VMEM((BW,), jnp.int32),
            pltpu.VMEM((BW, 128), jnp.float32),
            pltpu.SemaphoreType.DMA,
        ],
    )
    def sc_gather(idx_hbm, tbl_hbm, out_hbm, idx_v, row_v, rows_v, sem):
        wid = lax.axis_index("s") * 2 + lax.axis_index("c")
        pltpu.sync_copy(idx_hbm.at[pl.ds(wid * BW, BW)], idx_v)
        for k in range(BW // 16):
            row_v[pl.ds(k * 16, 16)] = lax.rem(idx_v[pl.ds(k * 16, 16)],
                                               jnp.int32(M))
        copies = []
        for j in range(CH):
            copies.append(
                pltpu.async_copy(tbl_hbm.at[row_v.at[pl.ds(j * 128, 128)]],
                                 rows_v.at[pl.ds(j * 128, 128)], sem))
        for c in copies:
            c.wait()
        pltpu.sync_copy(rows_v, out_hbm.at[pl.ds(wid * BW, BW)])

    return sc_gather


def _make_select_kernel(B, M, EMB, RB):
    def body(g_ref, n_ref, o_ref):
        n = n_ref[...]
        s32 = ((n >= M).astype(jnp.int32) + (n >= 2 * M).astype(jnp.int32)
               + (n >= 3 * M).astype(jnp.int32)) * EMB
        liota = lax.broadcasted_iota(jnp.int32, (RB, 128), 1)
        m = ((liota >= s32) & (liota < s32 + EMB)).astype(jnp.float32)
        gw = g_ref[...] * m
        ri = lax.broadcasted_iota(jnp.int32, (128, EMB), 0)
        ci = lax.broadcasted_iota(jnp.int32, (128, EMB), 1)
        fold = (lax.rem(ri, EMB) == ci).astype(jnp.float32)
        o_ref[...] = jnp.dot(gw, fold, preferred_element_type=jnp.float32)

    return pl.pallas_call(
        body,
        grid=(B // RB,),
        in_specs=[
            pl.BlockSpec((RB, 128), lambda i: (i, 0)),
            pl.BlockSpec((RB, 1), lambda i: (i, 0)),
        ],
        out_specs=pl.BlockSpec((RB, EMB), lambda i: (i, 0)),
        out_shape=jax.ShapeDtypeStruct((B, EMB), jnp.float32),
    )


def kernel(node_list, features, W, b):
    N, D = features.shape
    EMB = W.shape[0]
    B = node_list.shape[0]

    M = N // 4
    Q = 5000
    NB = M // Q

    ws = _make_window_kernel(N, M, Q, D, EMB, NB)(
        *([features] * 20), W.T, b.reshape(1, EMB))

    NW = 32
    H = B // 2
    CH = H // NW // 128
    outs = []
    for h in range(2):
        nl = lax.slice(node_list, (h * H,), ((h + 1) * H,))
        gathered = _make_sc_gather(H, M, NW, CH)(nl, ws)
        outs.append(_make_select_kernel(H, M, EMB, 4096)(
            gathered, nl.reshape(H, 1)))
    return jnp.concatenate(outs, axis=0)


# speedup vs baseline: 1.0316x; 1.0316x over previous
import functools

import jax
import jax.numpy as jnp
from jax import lax
from jax.experimental import pallas as pl
from jax.experimental.pallas import tpu as pltpu
from jax.experimental.pallas import tpu_sc as plsc

WIN = 33


def _make_window_kernel(N, M, Q, D, EMB, NB):
    def body(*refs):
        a_refs = refs[0:4]
        h_refs = refs[4:20]
        wt_ref, b_ref, o_ref = refs[20], refs[21], refs[22]
        QH = Q + 32
        parts = []
        for s in range(4):
            parts.append(a_refs[s][...])
            for j in range(4):
                parts.append(h_refs[4 * s + j][...])
        f_all = jnp.concatenate(parts, axis=0)
        h = jnp.dot(f_all, wt_ref[...], preferred_element_type=jnp.float32)
        g = jnp.tanh(h + b_ref[...])
        gp = jnp.concatenate([g[s * QH:(s + 1) * QH] for s in range(4)],
                             axis=1)
        a = gp[0:Q + 8] + gp[8:Q + 16] + gp[16:Q + 24] + gp[24:Q + 32]
        bb = a[0:Q + 7] + a[1:Q + 8]
        c = bb[0:Q + 5] + bb[2:Q + 7]
        d = c[0:Q] + c[4:Q + 4]
        o_ref[...] = (d + gp[32:Q + 32]) * (1.0 / WIN)

    in_specs = []
    for s in range(4):
        in_specs.append(
            pl.BlockSpec((Q, D), functools.partial(
                lambda s_, i: (s_ * (M // Q) + i, 0), s)))
    NH8 = N // 8
    for s in range(4):
        for j in range(4):
            in_specs.append(
                pl.BlockSpec((8, D), functools.partial(
                    lambda s_, j_, i: (
                        lax.rem(s_ * (M // 8) + (i + 1) * (Q // 8) + j_, NH8),
                        0), s, j)))
    in_specs.append(pl.BlockSpec((D, EMB), lambda i: (0, 0)))
    in_specs.append(pl.BlockSpec((1, EMB), lambda i: (0, 0)))

    return pl.pallas_call(
        body,
        grid=(NB,),
        in_specs=in_specs,
        out_specs=pl.BlockSpec((Q, 128), lambda i: (i, 0)),
        out_shape=jax.ShapeDtypeStruct((M, 128), jnp.float32),
    )


def _make_sc_gather(B, M, NW, CH):
    mesh = plsc.VectorSubcoreMesh(core_axis_name="c", subcore_axis_name="s")

    BW = CH * 128

    @functools.partial(
        pl.kernel,
        mesh=mesh,
        out_type=jax.ShapeDtypeStruct((B, 128), jnp.float32),
        scratch_types=[
            pltpu.VMEM((BW,), jnp.int32),
            pltpu.VMEM((BW,), jnp.int32),
            pltpu.VMEM((BW, 128), jnp.float32),
            pltpu.SemaphoreType.DMA,
        ],
    )
    def sc_gather(idx_hbm, tbl_hbm, out_hbm, idx_v, row_v, rows_v, sem):
        wid = lax.axis_index("s") * 2 + lax.axis_index("c")
        pltpu.sync_copy(idx_hbm.at[pl.ds(wid * BW, BW)], idx_v)
        for k in range(BW // 16):
            row_v[pl.ds(k * 16, 16)] = lax.rem(idx_v[pl.ds(k * 16, 16)],
                                               jnp.int32(M))
        copies = []
        for j in range(CH):
            copies.append(
                pltpu.async_copy(tbl_hbm.at[row_v.at[pl.ds(j * 128, 128)]],
                                 rows_v.at[pl.ds(j * 128, 128)], sem))
        for c in copies:
            c.wait()
        pltpu.sync_copy(rows_v, out_hbm.at[pl.ds(wid * BW, BW)])

    return sc_gather


def _make_select_kernel(B, M, EMB, RB):
    def body(g_ref, n_ref, o_ref):
        n = n_ref[...]
        s32 = ((n >= M).astype(jnp.int32) + (n >= 2 * M).astype(jnp.int32)
               + (n >= 3 * M).astype(jnp.int32)) * EMB
        liota = lax.broadcasted_iota(jnp.int32, (RB, 128), 1)
        m = ((liota >= s32) & (liota < s32 + EMB)).astype(jnp.float32)
        gw = g_ref[...] * m
        ri = lax.broadcasted_iota(jnp.int32, (128, EMB), 0)
        ci = lax.broadcasted_iota(jnp.int32, (128, EMB), 1)
        fold = (lax.rem(ri, EMB) == ci).astype(jnp.float32)
        o_ref[...] = jnp.dot(gw, fold, preferred_element_type=jnp.float32)

    return pl.pallas_call(
        body,
        grid=(B // RB,),
        in_specs=[
            pl.BlockSpec((RB, 128), lambda i: (i, 0)),
            pl.BlockSpec((RB, 1), lambda i: (i, 0)),
        ],
        out_specs=pl.BlockSpec((RB, EMB), lambda i: (i, 0)),
        out_shape=jax.ShapeDtypeStruct((B, EMB), jnp.float32),
    )


def kernel(node_list, features, W, b):
    N, D = features.shape
    EMB = W.shape[0]
    B = node_list.shape[0]

    M = N // 4
    Q = 5000
    NB = M // Q

    ws = _make_window_kernel(N, M, Q, D, EMB, NB)(
        *([features] * 20), W.T, b.reshape(1, EMB))

    NW = 32
    CH = B // NW // 128
    gathered = _make_sc_gather(B, M, NW, CH)(node_list, ws)

    return _make_select_kernel(B, M, EMB, 4096)(
        gathered, node_list.reshape(B, 1))
